# baseline (speedup 1.0000x reference)
import jax, jax.numpy as jnp
import numpy as np
from jax.experimental import pallas as pl

SIZES = (128.0, 256.0, 512.0)
RATIOS = (0.5, 1.0, 2.0)
PRE_NMS = 6000
POST_NMS = 1000
NMS_THRESH = 0.7
MIN_SIZE = 1.0


def _conv2d(x, w, b, pad):
    y = jax.lax.conv_general_dilated(x, w, window_strides=(1, 1), padding=[(pad, pad), (pad, pad)], dimension_numbers=('NCHW', 'OIHW', 'NCHW'))
    return y + b[None, :, None, None]


def _make_anchors(fh, fw, stride_y, stride_x):
    sizes = jnp.array(SIZES, dtype=jnp.float32)
    ratios = jnp.array(RATIOS, dtype=jnp.float32)
    hr = jnp.sqrt(ratios)
    wr = 1.0 / hr
    ws = (wr[:, None] * sizes[None, :]).reshape(-1)
    hs = (hr[:, None] * sizes[None, :]).reshape(-1)
    base = jnp.stack([-ws, -hs, ws, hs], axis=1) / 2.0
    shift_x = jnp.arange(fw, dtype=jnp.float32) * stride_x
    shift_y = jnp.arange(fh, dtype=jnp.float32) * stride_y
    yy, xx = jnp.meshgrid(shift_y, shift_x, indexing='ij')
    xs = xx.reshape(-1)
    ys = yy.reshape(-1)
    shifts = jnp.stack([xs, ys, xs, ys], axis=1)
    return (shifts[:, None, :] + base[None, :, :]).reshape(-1, 4)


def _decode(delta, boxes):
    dx = delta[:, 0]
    dy = delta[:, 1]
    dw = delta[:, 2]
    dh = delta[:, 3]
    lim = float(np.log(1000.0 / 16.0))
    dw = jnp.minimum(dw, lim)
    dh = jnp.minimum(dh, lim)
    w = boxes[:, 2] - boxes[:, 0]
    h = boxes[:, 3] - boxes[:, 1]
    cx = boxes[:, 0] + 0.5 * w
    cy = boxes[:, 1] + 0.5 * h
    pcx = dx * w + cx
    pcy = dy * h + cy
    pw = jnp.exp(dw) * w
    ph = jnp.exp(dh) * h
    return jnp.stack([pcx - 0.5 * pw, pcy - 0.5 * ph, pcx + 0.5 * pw, pcy + 0.5 * ph], axis=1)


def _iou_one(box, boxes):
    lt = jnp.maximum(box[:2], boxes[:, :2])
    rb = jnp.minimum(box[2:], boxes[:, 2:])
    wh = jnp.clip(rb - lt, 0.0)
    inter = wh[:, 0] * wh[:, 1]
    a1 = (box[2] - box[0]) * (box[3] - box[1])
    a2 = (boxes[:, 2] - boxes[:, 0]) * (boxes[:, 3] - boxes[:, 1])
    return inter / (a1 + a2 - inter)


def _nms_fixed(boxes, scores):
    n = scores.shape[0]
    ar = jnp.arange(n)

    def body(i, st):
        s, keep, valid = st
        idx = jnp.argmax(s)
        ok = s[idx] > -jnp.inf
        keep = keep.at[i].set(idx.astype(jnp.int32))
        valid = valid.at[i].set(ok)
        ious = _iou_one(boxes[idx], boxes)
        s = jnp.where((ious > NMS_THRESH) | (ar == idx), -jnp.inf, s)
        return (s, keep, valid)

    init = (scores, jnp.zeros((POST_NMS,), jnp.int32), jnp.zeros((POST_NMS,), bool))
    _, keep, valid = jax.lax.fori_loop(0, POST_NMS, body, init)
    return keep, valid


def _identity_kernel(x_ref, o_ref):
    o_ref[...] = x_ref[...]


def kernel(feature, image_shape, W1, b1, Wc, bc, Wb, bb):
    img_h = image_shape[0].astype(jnp.float32)
    img_w = image_shape[1].astype(jnp.float32)
    fh, fw = feature.shape[2], feature.shape[3]
    t = jax.nn.relu(_conv2d(feature, W1, b1, 1))
    cls = _conv2d(t, Wc, bc, 0)
    delta = _conv2d(t, Wb, bb, 0)
    cls = jnp.transpose(cls, (0, 2, 3, 1)).reshape(-1)
    delta = jnp.transpose(delta, (0, 2, 3, 1)).reshape(-1, 4)
    anchors = _make_anchors(fh, fw, img_h / fh, img_w / fw)
    pre = min(cls.shape[0], PRE_NMS)
    scores, top_idx = jax.lax.top_k(cls, pre)
    prop = _decode(delta[top_idx], anchors[top_idx])
    x1 = jnp.clip(prop[:, 0], 0.0, img_w)
    y1 = jnp.clip(prop[:, 1], 0.0, img_h)
    x2 = jnp.clip(prop[:, 2], 0.0, img_w)
    y2 = jnp.clip(prop[:, 3], 0.0, img_h)
    prop = jnp.stack([x1, y1, x2, y2], axis=1)
    small = ((x2 - x1) < MIN_SIZE) | ((y2 - y1) < MIN_SIZE)
    scores = jnp.where(small, -jnp.inf, scores)
    keep, valid = _nms_fixed(prop, scores)
    out = jnp.where(valid[:, None], prop[keep], 0.0)
    out = pl.pallas_call(
        _identity_kernel,
        out_shape=jax.ShapeDtypeStruct(out.shape, out.dtype),
    )(out)
    return out


# speedup vs baseline: 37.2394x; 37.2394x over previous
import jax
import jax.numpy as jnp
import numpy as np
from jax import lax
from jax.experimental import pallas as pl
from jax.experimental.pallas import tpu as pltpu

_SIZES = (128.0, 256.0, 512.0)
_RATIOS = (0.5, 1.0, 2.0)
_PRE_NMS = 6000
_POST_NMS = 1000
_NMS_THRESH = 0.7
_MIN_SIZE = 1.0
_LIM = float(np.log(1000.0 / 16.0))

_N_PAD = 6144
_N_CHUNKS = 48
_C = 128


def _conv2d(x, w, b, pad):
    y = lax.conv_general_dilated(
        x, w, window_strides=(1, 1), padding=[(pad, pad), (pad, pad)],
        dimension_numbers=('NCHW', 'OIHW', 'NCHW'))
    return y + b[None, :, None, None]


def _make_anchors(fh, fw, stride_y, stride_x):
    sizes = jnp.array(_SIZES, dtype=jnp.float32)
    ratios = jnp.array(_RATIOS, dtype=jnp.float32)
    hr = jnp.sqrt(ratios)
    wr = 1.0 / hr
    ws = (wr[:, None] * sizes[None, :]).reshape(-1)
    hs = (hr[:, None] * sizes[None, :]).reshape(-1)
    base = jnp.stack([-ws, -hs, ws, hs], axis=1) / 2.0
    shift_x = jnp.arange(fw, dtype=jnp.float32) * stride_x
    shift_y = jnp.arange(fh, dtype=jnp.float32) * stride_y
    yy, xx = jnp.meshgrid(shift_y, shift_x, indexing='ij')
    xs = xx.reshape(-1)
    ys = yy.reshape(-1)
    shifts = jnp.stack([xs, ys, xs, ys], axis=1)
    return (shifts[:, None, :] + base[None, :, :]).reshape(-1, 4)


def _nms_decode_kernel(d_ref, a_ref, img_ref, kept_ref, box_ref,
                       alive_ref, alivec_ref, mc_ref, m_ref, area_ref,
                       cnt_ref, conv_ref):
    f32 = jnp.float32
    img_w = img_ref[0:1, 0:1]
    img_h = img_ref[0:1, 1:2]

    dx = d_ref[0]
    dy = d_ref[1]
    dw = jnp.minimum(d_ref[2], _LIM)
    dh = jnp.minimum(d_ref[3], _LIM)
    aw = a_ref[2] - a_ref[0]
    ah = a_ref[3] - a_ref[1]
    acx = a_ref[0] + 0.5 * aw
    acy = a_ref[1] + 0.5 * ah
    pcx = dx * aw + acx
    pcy = dy * ah + acy
    pw = jnp.exp(dw) * aw
    ph = jnp.exp(dh) * ah
    x1 = jnp.clip(pcx - 0.5 * pw, 0.0, img_w)
    y1 = jnp.clip(pcy - 0.5 * ph, 0.0, img_h)
    x2 = jnp.clip(pcx + 0.5 * pw, 0.0, img_w)
    y2 = jnp.clip(pcy + 0.5 * ph, 0.0, img_h)
    box_ref[0] = x1
    box_ref[1] = y1
    box_ref[2] = x2
    box_ref[3] = y2
    small = ((x2 - x1) < _MIN_SIZE) | ((y2 - y1) < _MIN_SIZE)
    gidx = (lax.broadcasted_iota(jnp.int32, (_N_CHUNKS, _C), 0) * _C
            + lax.broadcasted_iota(jnp.int32, (_N_CHUNKS, _C), 1))
    valid = jnp.where(small | (gidx >= _PRE_NMS), 0.0, 1.0).astype(f32)
    area = (x2 - x1) * (y2 - y1)
    area_ref[...] = area

    alive_ref[...] = valid
    kept_ref[...] = jnp.zeros((_N_CHUNKS, _C), f32)
    cnt_ref[0, 0] = 0

    eye = (lax.broadcasted_iota(jnp.int32, (_C, _C), 0)
           == lax.broadcasted_iota(jnp.int32, (_C, _C), 1)).astype(f32)

    def col(row_2d):
        return lax.dot_general(eye, row_2d, (((1,), (1,)), ((), ())),
                               precision=lax.Precision.HIGHEST,
                               preferred_element_type=f32)

    x1r = x1[None]
    y1r = y1[None]
    x2r = x2[None]
    y2r = y2[None]
    arear = area[None]
    gj = gidx[None]

    def chunk_body(k, _):
        @pl.when(cnt_ref[0, 0] < _POST_NMS)
        def _():
            cx1 = col(box_ref[0, pl.ds(k, 1), :]).reshape(_C, 1, 1)
            cy1 = col(box_ref[1, pl.ds(k, 1), :]).reshape(_C, 1, 1)
            cx2 = col(box_ref[2, pl.ds(k, 1), :]).reshape(_C, 1, 1)
            cy2 = col(box_ref[3, pl.ds(k, 1), :]).reshape(_C, 1, 1)
            ca = col(area_ref[pl.ds(k, 1), :]).reshape(_C, 1, 1)
            ltx = jnp.maximum(cx1, x1r)
            lty = jnp.maximum(cy1, y1r)
            rbx = jnp.minimum(cx2, x2r)
            rby = jnp.minimum(cy2, y2r)
            iw = jnp.clip(rbx - ltx, 0.0)
            ih = jnp.clip(rby - lty, 0.0)
            inter = iw * ih
            iou = inter / (ca + arear - inter)
            gi = (k * _C + lax.broadcasted_iota(jnp.int32, (_C, 1, 1), 0))
            m = jnp.where((iou > _NMS_THRESH) & (gj > gi), 1.0, 0.0).astype(f32)
            m_ref[...] = m
            mc_ref[...] = m_ref[:, pl.ds(k, 1), :]

            alivec_ref[...] = alive_ref[pl.ds(k, 1), :]
            conv_ref[0, 0] = 0
            valid_c = alive_ref[pl.ds(k, 1), :]

            def fix_body(_, __):
                @pl.when(conv_ref[0, 0] == 0)
                def _():
                    ac = alivec_ref[...]
                    acol = col(ac).reshape(_C, 1, 1)
                    killed = jnp.max(acol * mc_ref[...], axis=0)
                    new = valid_c * (1.0 - killed)
                    changed = jnp.sum(jnp.abs(new - ac))
                    alivec_ref[...] = new
                    @pl.when(changed == 0.0)
                    def _():
                        conv_ref[0, 0] = 1
                return 0

            lax.fori_loop(0, _C, fix_body, 0)

            keptc = alivec_ref[...]
            kept_ref[pl.ds(k, 1), :] = keptc
            cnt_ref[0, 0] = cnt_ref[0, 0] + jnp.sum(keptc).astype(jnp.int32)
            kcol = col(keptc).reshape(_C, 1, 1)
            killed_all = jnp.max(kcol * m_ref[...], axis=0)
            alive_ref[...] = alive_ref[...] * (1.0 - killed_all)
        return 0

    lax.fori_loop(0, _N_CHUNKS, chunk_body, 0)


def _run_nms(d_planes, a_planes, img):
    return pl.pallas_call(
        _nms_decode_kernel,
        out_shape=(jax.ShapeDtypeStruct((_N_CHUNKS, _C), jnp.float32),
                   jax.ShapeDtypeStruct((4, _N_CHUNKS, _C), jnp.float32)),
        scratch_shapes=[
            pltpu.VMEM((_N_CHUNKS, _C), jnp.float32),
            pltpu.VMEM((1, _C), jnp.float32),
            pltpu.VMEM((_C, 1, _C), jnp.float32),
            pltpu.VMEM((_C, _N_CHUNKS, _C), jnp.float32),
            pltpu.VMEM((_N_CHUNKS, _C), jnp.float32),
            pltpu.SMEM((1, 1), jnp.int32),
            pltpu.SMEM((1, 1), jnp.int32),
        ],
    )(d_planes, a_planes, img)


def kernel(feature, image_shape, W1, b1, Wc, bc, Wb, bb):
    img_h = image_shape[0].astype(jnp.float32)
    img_w = image_shape[1].astype(jnp.float32)
    fh, fw = feature.shape[2], feature.shape[3]

    t = jax.nn.relu(_conv2d(feature, W1, b1, 1))
    cls = _conv2d(t, Wc, bc, 0)
    delta = _conv2d(t, Wb, bb, 0)
    cls = jnp.transpose(cls, (0, 2, 3, 1)).reshape(-1)
    delta = jnp.transpose(delta, (0, 2, 3, 1)).reshape(-1, 4)
    anchors = _make_anchors(fh, fw, img_h / fh, img_w / fw)
    _, top_idx = lax.top_k(cls, _PRE_NMS)
    d_sel = delta[top_idx]
    a_sel = anchors[top_idx]

    pad = _N_PAD - _PRE_NMS
    d_planes = jnp.pad(d_sel, ((0, pad), (0, 0))).T.reshape(4, _N_CHUNKS, _C)
    a_planes = jnp.pad(a_sel, ((0, pad), (0, 0))).T.reshape(4, _N_CHUNKS, _C)
    img = jnp.stack([img_w, img_h]).reshape(1, 2)

    kept, box_planes = _run_nms(d_planes, a_planes, img)

    keep = kept.reshape(-1)
    boxes = box_planes.reshape(4, _N_PAD).T
    pos = jnp.cumsum(keep).astype(jnp.int32) - 1
    take = (keep > 0.5) & (pos < _POST_NMS)
    tgt = jnp.where(take, pos, _POST_NMS)
    out = jnp.zeros((_POST_NMS, 4), jnp.float32).at[tgt].set(boxes, mode='drop')
    return out
